# baseline (speedup 1.0000x reference)
import functools

import jax
import jax.numpy as jnp
from jax import lax
from jax.experimental import pallas as pl
from jax.experimental.pallas import tpu as pltpu
from jax.experimental.pallas import tpu_sc as plsc

N_ENT = 100000
N_REL = 64
D = 128
K = 8
B = 1024
S = 3 * B

_NW = 32


@functools.lru_cache(maxsize=None)
def _sc_gather(M, Dw, dtype_name, chunk):
    dtype = jnp.dtype(dtype_name)
    bpw = M // _NW
    assert M % _NW == 0 and bpw % chunk == 0 and chunk % 8 == 0
    nch = bpw // chunk
    mesh = plsc.VectorSubcoreMesh(core_axis_name="c", subcore_axis_name="s")

    def body(table_hbm, idx_hbm, out_hbm, idx_v, rows_v, sem0, sem1):
        wid = lax.axis_index("s") * 2 + lax.axis_index("c")
        base = wid * bpw
        pltpu.sync_copy(idx_hbm.at[pl.ds(base, bpw)], idx_v)
        sems = (sem0, sem1)
        copies = [None] * nch

        def start(ci):
            return pltpu.async_copy(
                table_hbm.at[idx_v.at[pl.ds(ci * chunk, chunk)]],
                rows_v.at[ci % 2],
                sems[ci % 2],
            )

        copies[0] = start(0)
        for ci in range(nch):
            copies[ci].wait()
            if ci + 1 < nch:
                copies[ci + 1] = start(ci + 1)
            pltpu.sync_copy(rows_v.at[ci % 2],
                            out_hbm.at[pl.ds(base + ci * chunk, chunk)])

    return pl.kernel(
        body,
        out_type=jax.ShapeDtypeStruct((M, Dw), dtype),
        mesh=mesh,
        scratch_types=[
            pltpu.VMEM((bpw,), jnp.int32),
            pltpu.VMEM((2, chunk, Dw), dtype),
            pltpu.SemaphoreType.DMA,
            pltpu.SemaphoreType.DMA,
        ],
    )


_SB = S // _NW


@functools.lru_cache(maxsize=None)
def _sc_stage_a():
    f32, i32 = jnp.float32, jnp.int32
    mesh = plsc.VectorSubcoreMesh(core_axis_name="c", subcore_axis_name="s")

    _dn = lax.GatherDimensionNumbers(offset_dims=(), collapsed_slice_dims=(0,),
                                     start_index_map=(0,))

    def vperm(vec, idx):
        return lax.gather(vec, idx[:, None], _dn, (1,),
                          mode=lax.GatherScatterMode.PROMISE_IN_BOUNDS)

    def word_idx(src_ref, n, dst_ref):
        iota = lax.broadcasted_iota(i32, (16,), 0)
        half = iota >> 3
        low = iota & 7
        for g in range(n // 16):
            s16 = src_ref[pl.ds(g * 16, 16)]
            for q in range(8):
                srep = vperm(s16, half + 2 * q)
                dst_ref[pl.ds(g * 128 + q * 16, 16)] = srep * K + low

    def body(adj_e, adj_r, embs, seeds, hq_o, t1_o, t2_o, r0_o, r1_o,
             sv, iw, e1v, r0v, iw2, e2v, r1v, fbuf, tbuf,
             sema, semh, sem0, sem1):
        w = lax.axis_index("s") * 2 + lax.axis_index("c")
        s0 = w * _SB
        pltpu.sync_copy(seeds.at[pl.ds(s0, _SB)], sv)
        word_idx(sv, _SB, iw)
        cp_e = pltpu.async_copy(adj_e.at[iw], e1v, sema)
        cp_r = pltpu.async_copy(adj_r.at[iw], r0v, semh)
        cp_e.wait()
        cp_r.wait()
        pltpu.sync_copy(r0v, r0_o.at[pl.ds(s0 * K, _SB * K)])
        t1_cp0 = pltpu.async_copy(embs.at[e1v.at[pl.ds(0, 192)]],
                                  fbuf.at[0], sem0)
        word_idx(e1v, _SB * K, iw2)
        cp_e = pltpu.async_copy(adj_e.at[iw2], e2v, sema)
        cp_r = pltpu.async_copy(adj_r.at[iw2], r1v, sem1)
        cp_e.wait()
        cp_r.wait()
        pltpu.sync_copy(r1v, r1_o.at[pl.ds(s0 * K * K, _SB * K * K)])
        t2_cp0 = pltpu.async_copy(embs.at[e2v.at[pl.ds(0, 192)]],
                                  tbuf.at[0], semh)

        def t2chunk(i, c):
            c1 = pltpu.async_copy(
                embs.at[e2v.at[pl.ds(i * 384 + 192, 192)]], tbuf.at[1], sema)
            pltpu.make_async_copy(
                embs.at[e2v.at[pl.ds(0, 192)]], tbuf.at[0], semh).wait()
            pltpu.sync_copy(tbuf.at[0],
                            t2_o.at[pl.ds(s0 * K * K + i * 384, 192)])
            c0n = pltpu.async_copy(
                embs.at[e2v.at[pl.ds((i + 1) * 384 % (_SB * K * K), 192)]],
                tbuf.at[0], semh)
            c1.wait()
            pltpu.sync_copy(tbuf.at[1],
                            t2_o.at[pl.ds(s0 * K * K + i * 384 + 192, 192)])
            return c
        lax.fori_loop(0, _SB * K * K // 384, t2chunk, 0)
        pltpu.make_async_copy(
            embs.at[e2v.at[pl.ds(0, 192)]], tbuf.at[0], semh).wait()
        sems = (sem0, sem1)
        cps = [t1_cp0, None, None, None]
        for ci in range(4):
            if ci + 1 < 4:
                cps[ci + 1] = pltpu.async_copy(
                    embs.at[e1v.at[pl.ds((ci + 1) * 192, 192)]],
                    fbuf.at[(ci + 1) % 2], sems[(ci + 1) % 2])
            cps[ci].wait()
            pltpu.sync_copy(fbuf.at[ci % 2],
                            t1_o.at[pl.ds(s0 * K + ci * 192, 192)])
        pltpu.async_copy(embs.at[sv], fbuf.at[0, pl.ds(0, _SB)], sem0).wait()
        pltpu.sync_copy(fbuf.at[0, pl.ds(0, _SB)], hq_o.at[pl.ds(s0, _SB)])

    return pl.kernel(
        body,
        out_type=(
            jax.ShapeDtypeStruct((S, D), f32),
            jax.ShapeDtypeStruct((S * K, D), f32),
            jax.ShapeDtypeStruct((S * K * K, D), f32),
            jax.ShapeDtypeStruct((S * K,), i32),
            jax.ShapeDtypeStruct((S * K * K,), i32),
        ),
        mesh=mesh,
        scratch_types=[
            pltpu.VMEM((_SB,), i32),
            pltpu.VMEM((_SB * K,), i32),
            pltpu.VMEM((_SB * K,), i32),
            pltpu.VMEM((_SB * K,), i32),
            pltpu.VMEM((_SB * K * K,), i32),
            pltpu.VMEM((_SB * K * K,), i32),
            pltpu.VMEM((_SB * K * K,), i32),
            pltpu.VMEM((2, 192, D), f32),
            pltpu.VMEM((2, 192, D), f32),
            pltpu.SemaphoreType.DMA,
            pltpu.SemaphoreType.DMA,
            pltpu.SemaphoreType.DMA,
            pltpu.SemaphoreType.DMA,
        ],
    )


def _dense_body(hq_ref, t1_ref, t2_ref, r0_ref, r1_ref, rel_ref, A1_ref,
                A2_ref, A3_ref, Wxw_ref, Wxb_ref, W1w_ref, W1b_ref, W2w_ref,
                W2b_ref, out_ref):
    f32 = jnp.float32
    R = hq_ref.shape[0]
    ones_col = jnp.ones((D, 1), f32)

    def row_scale(v):
        n2 = jnp.dot(v * v, ones_col, preferred_element_type=f32)
        return jnp.minimum(lax.rsqrt(n2), 1.0)

    def norm_rows(v):
        return v * row_scale(v)

    def leaky(x):
        return jnp.where(x > 0, x, 0.2 * x)

    rel_n = norm_rows(rel_ref[...])
    A1h = A1_ref[0:D, :]
    A1r = A1_ref[D:2 * D, :]
    Brel = jnp.dot(rel_n, A1r, preferred_element_type=f32)
    A2 = A2_ref[...]
    A3c = A3_ref[...]
    Wxw = Wxw_ref[...]
    Wxb = Wxb_ref[...]
    W1f = W1w_ref[0:D, :] + W1w_ref[D:2 * D, :]
    W2f = W2w_ref[0:D, :] + W2w_ref[D:2 * D, :]
    W1b = W1b_ref[...]
    W2b = W2b_ref[...]

    hq = norm_rows(hq_ref[...])
    t1 = norm_rows(t1_ref[...])
    t2 = t2_ref[...]
    ts2 = row_scale(t2)

    def hop(h, tt, tscale, ridx, k):
        a = jnp.dot(h, A1h, preferred_element_type=f32)
        arep = jnp.broadcast_to(a.reshape(R, 1, D), (R, k, D)).reshape(R * k, D)
        oh = (ridx == lax.broadcasted_iota(jnp.int32, (R * k, N_REL), 1)
              ).astype(f32)
        b = jnp.dot(oh, Brel, preferred_element_type=f32)
        v = jnp.maximum(arep + b, 0.0)
        w = jnp.maximum(jnp.dot(v, A2, preferred_element_type=f32), 0.0)
        aw = jnp.dot(w, A3c, preferred_element_type=f32)
        ex = jnp.exp(1.0 / (1.0 + jnp.exp(-aw)))
        den = jnp.sum(ex.reshape(R, k, 1), axis=1)
        exw = ex * tscale if tscale is not None else ex
        num = jnp.sum((exw * tt).reshape(R, k, D), axis=1)
        e = num / den
        return leaky(jnp.dot(e, Wxw, preferred_element_type=f32) + Wxb)

    def agg(h, g):
        return (leaky(jnp.dot(h + g, W1f, preferred_element_type=f32) + W1b)
                + leaky(jnp.dot(h * g, W2f, preferred_element_type=f32) + W2b))

    g1 = hop(hq, t1, None, r0_ref[...], K)
    o1 = agg(hq, g1)
    hsum = jnp.sum(t1.reshape(R, K, D), axis=1)
    g2 = hop(hsum, t2, ts2, r1_ref[...], K * K)
    o2 = agg(hsum, g2)
    out_ref[:, 0:256] = o2
    out_ref[:, 256:512] = o1
    out_ref[:, 512:640] = hq


@functools.lru_cache(maxsize=None)
def _dense_call(R, interpret=False):
    grid = (S // R,)
    const = lambda i: (0, 0)
    row = lambda i: (i, 0)
    return pl.pallas_call(
        _dense_body,
        grid=grid,
        in_specs=[
            pl.BlockSpec((R, D), row),
            pl.BlockSpec((R * K, D), row),
            pl.BlockSpec((R * K * K, D), row),
            pl.BlockSpec((R * K, 1), row),
            pl.BlockSpec((R * K * K, 1), row),
            pl.BlockSpec((N_REL, D), const),
            pl.BlockSpec((2 * D, D), const),
            pl.BlockSpec((D, D), const),
            pl.BlockSpec((D, 1), const),
            pl.BlockSpec((D, D), const),
            pl.BlockSpec((1, D), const),
            pl.BlockSpec((2 * D, 2 * D), const),
            pl.BlockSpec((1, 2 * D), const),
            pl.BlockSpec((2 * D, 2 * D), const),
            pl.BlockSpec((1, 2 * D), const),
        ],
        out_specs=pl.BlockSpec((R, 640), row),
        out_shape=jax.ShapeDtypeStruct((S, 640), jnp.float32),
        interpret=interpret,
    )


def kernel(entity_embs, relation_embs, A1, A2, A3, Wx_w, Wx_b, W1_w, W1_b,
           W2_w, W2_b, u1, u2, c, adj_entity, adj_relation):
    i32 = jnp.int32
    adj_e = adj_entity.astype(i32).reshape(-1)
    adj_r = adj_relation.astype(i32).reshape(-1)
    seeds = jnp.concatenate([c, u1, u2]).astype(i32)

    hq_rows, t1_rows, t2_rows, r0, r1 = _sc_stage_a()(adj_e, adj_r,
                                                      entity_embs, seeds)

    out640 = _dense_call(128)(
        hq_rows, t1_rows, t2_rows, r0.reshape(S * K, 1),
        r1.reshape(S * K * K, 1),
        relation_embs, A1, A2, A3, Wx_w, Wx_b.reshape(1, D),
        W1_w, W1_b.reshape(1, 2 * D), W2_w, W2_b.reshape(1, 2 * D))

    return jnp.concatenate([out640[0:B], out640[B:2 * B], out640[2 * B:]],
                           axis=-1)


# speedup vs baseline: 1.0169x; 1.0169x over previous
import functools

import jax
import jax.numpy as jnp
from jax import lax
from jax.experimental import pallas as pl
from jax.experimental.pallas import tpu as pltpu
from jax.experimental.pallas import tpu_sc as plsc

N_ENT = 100000
N_REL = 64
D = 128
K = 8
B = 1024
S = 3 * B

_NW = 32


@functools.lru_cache(maxsize=None)
def _sc_gather(M, Dw, dtype_name, chunk):
    dtype = jnp.dtype(dtype_name)
    bpw = M // _NW
    assert M % _NW == 0 and bpw % chunk == 0 and chunk % 8 == 0
    nch = bpw // chunk
    mesh = plsc.VectorSubcoreMesh(core_axis_name="c", subcore_axis_name="s")

    NB = 3

    def body(table_hbm, idx_hbm, out_hbm, idx_v, rows_v,
             g0, g1, g2, w0, w1, w2):
        wid = lax.axis_index("s") * 2 + lax.axis_index("c")
        base = wid * bpw
        pltpu.sync_copy(idx_hbm.at[pl.ds(base, bpw)], idx_v)
        gsem = (g0, g1, g2)
        wsem = (w0, w1, w2)

        def gstart(ci):
            return pltpu.async_copy(
                table_hbm.at[idx_v.at[pl.ds(ci * chunk, chunk)]],
                rows_v.at[ci % NB], gsem[ci % NB])

        gcop = [None] * nch
        wcop = [None] * nch
        for ci in range(min(NB, nch)):
            gcop[ci] = gstart(ci)
        for ci in range(nch):
            gcop[ci].wait()
            wcop[ci] = pltpu.async_copy(
                rows_v.at[ci % NB],
                out_hbm.at[pl.ds(base + ci * chunk, chunk)], wsem[ci % NB])
            nxt = ci + NB
            if nxt < nch:
                wcop[ci].wait()
                gcop[nxt] = gstart(nxt)
        for ci in range(nch):
            if ci + NB >= nch:
                wcop[ci].wait()

    return pl.kernel(
        body,
        out_type=jax.ShapeDtypeStruct((M, Dw), dtype),
        mesh=mesh,
        scratch_types=[
            pltpu.VMEM((bpw,), jnp.int32),
            pltpu.VMEM((NB, chunk, Dw), dtype),
            pltpu.SemaphoreType.DMA,
            pltpu.SemaphoreType.DMA,
            pltpu.SemaphoreType.DMA,
            pltpu.SemaphoreType.DMA,
            pltpu.SemaphoreType.DMA,
            pltpu.SemaphoreType.DMA,
        ],
    )


_SB = S // _NW


@functools.lru_cache(maxsize=None)
def _sc_stage_a():
    f32, i32 = jnp.float32, jnp.int32
    mesh = plsc.VectorSubcoreMesh(core_axis_name="c", subcore_axis_name="s")

    _dn = lax.GatherDimensionNumbers(offset_dims=(), collapsed_slice_dims=(0,),
                                     start_index_map=(0,))

    def vperm(vec, idx):
        return lax.gather(vec, idx[:, None], _dn, (1,),
                          mode=lax.GatherScatterMode.PROMISE_IN_BOUNDS)

    def word_idx(src_ref, n, dst_ref):
        iota = lax.broadcasted_iota(i32, (16,), 0)
        half = iota >> 3
        low = iota & 7
        for g in range(n // 16):
            s16 = src_ref[pl.ds(g * 16, 16)]
            for q in range(8):
                srep = vperm(s16, half + 2 * q)
                dst_ref[pl.ds(g * 128 + q * 16, 16)] = srep * K + low

    def body(adj_e, adj_r, embs, seeds, hq_o, t1_o, r0_o, r1_o, e2_o,
             sv, iw, e1v, r0v, iw2, e2v, r1v, fbuf,
             sema, semh, sem0, sem1):
        w = lax.axis_index("s") * 2 + lax.axis_index("c")
        s0 = w * _SB
        pltpu.sync_copy(seeds.at[pl.ds(s0, _SB)], sv)
        word_idx(sv, _SB, iw)
        cp_e = pltpu.async_copy(adj_e.at[iw], e1v, sema)
        cp_r = pltpu.async_copy(adj_r.at[iw], r0v, semh)
        cp_e.wait()
        cp_r.wait()
        pltpu.sync_copy(r0v, r0_o.at[pl.ds(s0 * K, _SB * K)])
        t1_cp0 = pltpu.async_copy(embs.at[e1v.at[pl.ds(0, 192)]],
                                  fbuf.at[0], sem0)
        word_idx(e1v, _SB * K, iw2)
        cp_e = pltpu.async_copy(adj_e.at[iw2], e2v, sema)
        cp_r = pltpu.async_copy(adj_r.at[iw2], r1v, sem1)
        cp_e.wait()
        cp_r.wait()
        pltpu.sync_copy(r1v, r1_o.at[pl.ds(s0 * K * K, _SB * K * K)])
        pltpu.sync_copy(e2v, e2_o.at[pl.ds(s0 * K * K, _SB * K * K)])
        sems = (sem0, sem1)
        cps = [t1_cp0, None, None, None]
        for ci in range(4):
            if ci + 1 < 4:
                cps[ci + 1] = pltpu.async_copy(
                    embs.at[e1v.at[pl.ds((ci + 1) * 192, 192)]],
                    fbuf.at[(ci + 1) % 2], sems[(ci + 1) % 2])
            cps[ci].wait()
            pltpu.sync_copy(fbuf.at[ci % 2],
                            t1_o.at[pl.ds(s0 * K + ci * 192, 192)])
        pltpu.async_copy(embs.at[sv], fbuf.at[0, pl.ds(0, _SB)], sem0).wait()
        pltpu.sync_copy(fbuf.at[0, pl.ds(0, _SB)], hq_o.at[pl.ds(s0, _SB)])

    return pl.kernel(
        body,
        out_type=(
            jax.ShapeDtypeStruct((S, D), f32),
            jax.ShapeDtypeStruct((S * K, D), f32),
            jax.ShapeDtypeStruct((S * K,), i32),
            jax.ShapeDtypeStruct((S * K * K,), i32),
            jax.ShapeDtypeStruct((S * K * K,), i32),
        ),
        mesh=mesh,
        scratch_types=[
            pltpu.VMEM((_SB,), i32),
            pltpu.VMEM((_SB * K,), i32),
            pltpu.VMEM((_SB * K,), i32),
            pltpu.VMEM((_SB * K,), i32),
            pltpu.VMEM((_SB * K * K,), i32),
            pltpu.VMEM((_SB * K * K,), i32),
            pltpu.VMEM((_SB * K * K,), i32),
            pltpu.VMEM((2, 192, D), f32),
            pltpu.SemaphoreType.DMA,
            pltpu.SemaphoreType.DMA,
            pltpu.SemaphoreType.DMA,
            pltpu.SemaphoreType.DMA,
        ],
    )


def _dense_body(hq_ref, t1_ref, t2_ref, r0_ref, r1_ref, rel_ref, A1_ref,
                A2_ref, A3_ref, Wxw_ref, Wxb_ref, W1w_ref, W1b_ref, W2w_ref,
                W2b_ref, out_ref):
    f32 = jnp.float32
    R = hq_ref.shape[0]
    ones_col = jnp.ones((D, 1), f32)

    def row_scale(v):
        n2 = jnp.dot(v * v, ones_col, preferred_element_type=f32)
        return jnp.minimum(lax.rsqrt(n2), 1.0)

    def norm_rows(v):
        return v * row_scale(v)

    def leaky(x):
        return jnp.where(x > 0, x, 0.2 * x)

    rel_n = norm_rows(rel_ref[...])
    A1h = A1_ref[0:D, :]
    A1r = A1_ref[D:2 * D, :]
    Brel = jnp.dot(rel_n, A1r, preferred_element_type=f32)
    A2 = A2_ref[...]
    A3c = A3_ref[...]
    Wxw = Wxw_ref[...]
    Wxb = Wxb_ref[...]
    W1f = W1w_ref[0:D, :] + W1w_ref[D:2 * D, :]
    W2f = W2w_ref[0:D, :] + W2w_ref[D:2 * D, :]
    W1b = W1b_ref[...]
    W2b = W2b_ref[...]

    hq = norm_rows(hq_ref[...])
    t1 = norm_rows(t1_ref[...])
    t2 = t2_ref[...]
    ts2 = row_scale(t2)

    def hop(h, tt, tscale, ridx, k):
        a = jnp.dot(h, A1h, preferred_element_type=f32)
        arep = jnp.broadcast_to(a.reshape(R, 1, D), (R, k, D)).reshape(R * k, D)
        oh = (ridx == lax.broadcasted_iota(jnp.int32, (R * k, N_REL), 1)
              ).astype(f32)
        b = jnp.dot(oh, Brel, preferred_element_type=f32)
        v = jnp.maximum(arep + b, 0.0)
        w = jnp.maximum(jnp.dot(v, A2, preferred_element_type=f32), 0.0)
        aw = jnp.dot(w, A3c, preferred_element_type=f32)
        ex = jnp.exp(1.0 / (1.0 + jnp.exp(-aw)))
        den = jnp.sum(ex.reshape(R, k, 1), axis=1)
        exw = ex * tscale if tscale is not None else ex
        num = jnp.sum((exw * tt).reshape(R, k, D), axis=1)
        e = num / den
        return leaky(jnp.dot(e, Wxw, preferred_element_type=f32) + Wxb)

    def agg(h, g):
        return (leaky(jnp.dot(h + g, W1f, preferred_element_type=f32) + W1b)
                + leaky(jnp.dot(h * g, W2f, preferred_element_type=f32) + W2b))

    g1 = hop(hq, t1, None, r0_ref[...], K)
    o1 = agg(hq, g1)
    hsum = jnp.sum(t1.reshape(R, K, D), axis=1)
    g2 = hop(hsum, t2, ts2, r1_ref[...], K * K)
    o2 = agg(hsum, g2)
    out_ref[:, 0:256] = o2
    out_ref[:, 256:512] = o1
    out_ref[:, 512:640] = hq


@functools.lru_cache(maxsize=None)
def _dense_call(R, interpret=False):
    grid = (S // R,)
    const = lambda i: (0, 0)
    row = lambda i: (i, 0)
    return pl.pallas_call(
        _dense_body,
        grid=grid,
        in_specs=[
            pl.BlockSpec((R, D), row),
            pl.BlockSpec((R * K, D), row),
            pl.BlockSpec((R * K * K, D), row),
            pl.BlockSpec((R * K, 1), row),
            pl.BlockSpec((R * K * K, 1), row),
            pl.BlockSpec((N_REL, D), const),
            pl.BlockSpec((2 * D, D), const),
            pl.BlockSpec((D, D), const),
            pl.BlockSpec((D, 1), const),
            pl.BlockSpec((D, D), const),
            pl.BlockSpec((1, D), const),
            pl.BlockSpec((2 * D, 2 * D), const),
            pl.BlockSpec((1, 2 * D), const),
            pl.BlockSpec((2 * D, 2 * D), const),
            pl.BlockSpec((1, 2 * D), const),
        ],
        out_specs=pl.BlockSpec((R, 640), row),
        out_shape=jax.ShapeDtypeStruct((S, 640), jnp.float32),
        interpret=interpret,
    )


def kernel(entity_embs, relation_embs, A1, A2, A3, Wx_w, Wx_b, W1_w, W1_b,
           W2_w, W2_b, u1, u2, c, adj_entity, adj_relation):
    i32 = jnp.int32
    adj_e = adj_entity.astype(i32).reshape(-1)
    adj_r = adj_relation.astype(i32).reshape(-1)
    seeds = jnp.concatenate([c, u1, u2]).astype(i32)

    hq_rows, t1_rows, r0, r1, e2 = _sc_stage_a()(adj_e, adj_r,
                                                 entity_embs, seeds)
    t2_rows = _sc_gather(S * K * K, D, "float32", 256)(entity_embs, e2)

    out640 = _dense_call(128)(
        hq_rows, t1_rows, t2_rows, r0.reshape(S * K, 1),
        r1.reshape(S * K * K, 1),
        relation_embs, A1, A2, A3, Wx_w, Wx_b.reshape(1, D),
        W1_w, W1_b.reshape(1, 2 * D), W2_w, W2_b.reshape(1, 2 * D))

    return jnp.concatenate([out640[0:B], out640[B:2 * B], out640[2 * B:]],
                           axis=-1)


# speedup vs baseline: 1.0327x; 1.0156x over previous
import functools

import jax
import jax.numpy as jnp
from jax import lax
from jax.experimental import pallas as pl
from jax.experimental.pallas import tpu as pltpu
from jax.experimental.pallas import tpu_sc as plsc

N_ENT = 100000
N_REL = 64
D = 128
K = 8
B = 1024
S = 3 * B

_NW = 32


@functools.lru_cache(maxsize=None)
def _sc_gather(M, Dw, dtype_name, chunk):
    dtype = jnp.dtype(dtype_name)
    bpw = M // _NW
    assert M % _NW == 0 and bpw % chunk == 0 and chunk % 8 == 0
    nch = bpw // chunk
    mesh = plsc.VectorSubcoreMesh(core_axis_name="c", subcore_axis_name="s")

    NB = 3

    def body(table_hbm, idx_hbm, out_hbm, idx_v, rows_v,
             g0, g1, g2, w0, w1, w2):
        wid = lax.axis_index("s") * 2 + lax.axis_index("c")
        base = wid * bpw
        pltpu.sync_copy(idx_hbm.at[pl.ds(base, bpw)], idx_v)
        gsem = (g0, g1, g2)
        wsem = (w0, w1, w2)

        def gstart(ci):
            return pltpu.async_copy(
                table_hbm.at[idx_v.at[pl.ds(ci * chunk, chunk)]],
                rows_v.at[ci % NB], gsem[ci % NB])

        gcop = [None] * nch
        wcop = [None] * nch
        for ci in range(min(NB, nch)):
            gcop[ci] = gstart(ci)
        for ci in range(nch):
            gcop[ci].wait()
            wcop[ci] = pltpu.async_copy(
                rows_v.at[ci % NB],
                out_hbm.at[pl.ds(base + ci * chunk, chunk)], wsem[ci % NB])
            nxt = ci + NB
            if nxt < nch:
                wcop[ci].wait()
                gcop[nxt] = gstart(nxt)
        for ci in range(nch):
            if ci + NB >= nch:
                wcop[ci].wait()

    return pl.kernel(
        body,
        out_type=jax.ShapeDtypeStruct((M, Dw), dtype),
        mesh=mesh,
        scratch_types=[
            pltpu.VMEM((bpw,), jnp.int32),
            pltpu.VMEM((NB, chunk, Dw), dtype),
            pltpu.SemaphoreType.DMA,
            pltpu.SemaphoreType.DMA,
            pltpu.SemaphoreType.DMA,
            pltpu.SemaphoreType.DMA,
            pltpu.SemaphoreType.DMA,
            pltpu.SemaphoreType.DMA,
        ],
    )


_SB = S // _NW


@functools.lru_cache(maxsize=None)
def _sc_stage_a():
    f32, i32 = jnp.float32, jnp.int32
    mesh = plsc.VectorSubcoreMesh(core_axis_name="c", subcore_axis_name="s")

    _dn = lax.GatherDimensionNumbers(offset_dims=(), collapsed_slice_dims=(0,),
                                     start_index_map=(0,))

    def vperm(vec, idx):
        return lax.gather(vec, idx[:, None], _dn, (1,),
                          mode=lax.GatherScatterMode.PROMISE_IN_BOUNDS)

    def word_idx(src_ref, n, dst_ref):
        iota = lax.broadcasted_iota(i32, (16,), 0)
        half = iota >> 3
        low = iota & 7
        for g in range(n // 16):
            s16 = src_ref[pl.ds(g * 16, 16)]
            for q in range(8):
                srep = vperm(s16, half + 2 * q)
                dst_ref[pl.ds(g * 128 + q * 16, 16)] = srep * K + low

    def body(adj_e, adj_r, embs, seeds, hq_o, t1_o, r0_o, r1_o, e2_o,
             sv, iw, e1v, r0v, iw2, e2v, r1v, fbuf,
             sema, semh, sem0, sem1):
        w = lax.axis_index("s") * 2 + lax.axis_index("c")
        s0 = w * _SB
        pltpu.sync_copy(seeds.at[pl.ds(s0, _SB)], sv)
        word_idx(sv, _SB, iw)
        cp_e = pltpu.async_copy(adj_e.at[iw], e1v, sema)
        cp_r = pltpu.async_copy(adj_r.at[iw], r0v, semh)
        cp_e.wait()
        cp_r.wait()
        pltpu.sync_copy(r0v, r0_o.at[pl.ds(s0 * K, _SB * K)])
        t1_cp0 = pltpu.async_copy(embs.at[e1v.at[pl.ds(0, 192)]],
                                  fbuf.at[0], sem0)
        word_idx(e1v, _SB * K, iw2)
        cp_e = pltpu.async_copy(adj_e.at[iw2], e2v, sema)
        cp_r = pltpu.async_copy(adj_r.at[iw2], r1v, sem1)
        cp_e.wait()
        cp_r.wait()
        pltpu.sync_copy(r1v, r1_o.at[pl.ds(s0 * K * K, _SB * K * K)])
        pltpu.sync_copy(e2v, e2_o.at[pl.ds(s0 * K * K, _SB * K * K)])
        sems = (sem0, sem1)
        cps = [t1_cp0, None, None, None]
        for ci in range(4):
            if ci + 1 < 4:
                cps[ci + 1] = pltpu.async_copy(
                    embs.at[e1v.at[pl.ds((ci + 1) * 192, 192)]],
                    fbuf.at[(ci + 1) % 2], sems[(ci + 1) % 2])
            cps[ci].wait()
            pltpu.sync_copy(fbuf.at[ci % 2],
                            t1_o.at[pl.ds(s0 * K + ci * 192, 192)])
        pltpu.async_copy(embs.at[sv], fbuf.at[0, pl.ds(0, _SB)], sem0).wait()
        pltpu.sync_copy(fbuf.at[0, pl.ds(0, _SB)], hq_o.at[pl.ds(s0, _SB)])

    return pl.kernel(
        body,
        out_type=(
            jax.ShapeDtypeStruct((S, D), f32),
            jax.ShapeDtypeStruct((S * K, D), f32),
            jax.ShapeDtypeStruct((S * K,), i32),
            jax.ShapeDtypeStruct((S * K * K,), i32),
            jax.ShapeDtypeStruct((S * K * K,), i32),
        ),
        mesh=mesh,
        scratch_types=[
            pltpu.VMEM((_SB,), i32),
            pltpu.VMEM((_SB * K,), i32),
            pltpu.VMEM((_SB * K,), i32),
            pltpu.VMEM((_SB * K,), i32),
            pltpu.VMEM((_SB * K * K,), i32),
            pltpu.VMEM((_SB * K * K,), i32),
            pltpu.VMEM((_SB * K * K,), i32),
            pltpu.VMEM((2, 192, D), f32),
            pltpu.SemaphoreType.DMA,
            pltpu.SemaphoreType.DMA,
            pltpu.SemaphoreType.DMA,
            pltpu.SemaphoreType.DMA,
        ],
    )


def _dense_body(hq_ref, t1_ref, t2_ref, r0_ref, r1_ref, rel_ref, A1_ref,
                A2_ref, A3_ref, Wxw_ref, Wxb_ref, W1w_ref, W1b_ref, W2w_ref,
                W2b_ref, out_ref):
    f32 = jnp.float32
    R = hq_ref.shape[0]
    ones_col = jnp.ones((D, 1), f32)

    def row_scale(v):
        n2 = jnp.dot(v * v, ones_col, preferred_element_type=f32)
        return jnp.minimum(lax.rsqrt(n2), 1.0)

    def norm_rows(v):
        return v * row_scale(v)

    def leaky(x):
        return jnp.where(x > 0, x, 0.2 * x)

    rel_n = norm_rows(rel_ref[...])
    A1h = A1_ref[0:D, :]
    A1r = A1_ref[D:2 * D, :]
    Brel = jnp.dot(rel_n, A1r, preferred_element_type=f32)
    A2 = A2_ref[...]
    A3c = A3_ref[...]
    Wxw = Wxw_ref[...]
    Wxb = Wxb_ref[...]
    W1f = W1w_ref[0:D, :] + W1w_ref[D:2 * D, :]
    W2f = W2w_ref[0:D, :] + W2w_ref[D:2 * D, :]
    W1b = W1b_ref[...]
    W2b = W2b_ref[...]

    hq = norm_rows(hq_ref[...])
    t1 = norm_rows(t1_ref[...])
    t2 = t2_ref[...]
    ts2 = row_scale(t2)

    def hop(h, tt, tscale, ridx, k):
        a = jnp.dot(h, A1h, preferred_element_type=f32)
        arep = jnp.broadcast_to(a.reshape(R, 1, D), (R, k, D)).reshape(R * k, D)
        oh = (ridx == lax.broadcasted_iota(jnp.int32, (R * k, N_REL), 1)
              ).astype(f32)
        b = jnp.dot(oh, Brel, preferred_element_type=f32)
        v = jnp.maximum(arep + b, 0.0)
        w = jnp.maximum(jnp.dot(v, A2, preferred_element_type=f32), 0.0)
        aw = jnp.dot(w, A3c, preferred_element_type=f32)
        ex = jnp.exp(1.0 / (1.0 + jnp.exp(-aw)))
        den = jnp.sum(ex.reshape(R, k, 1), axis=1)
        exw = ex * tscale if tscale is not None else ex
        num = jnp.sum((exw * tt).reshape(R, k, D), axis=1)
        e = num / den
        return leaky(jnp.dot(e, Wxw, preferred_element_type=f32) + Wxb)

    def agg(h, g):
        return (leaky(jnp.dot(h + g, W1f, preferred_element_type=f32) + W1b)
                + leaky(jnp.dot(h * g, W2f, preferred_element_type=f32) + W2b))

    g1 = hop(hq, t1, None, r0_ref[...], K)
    o1 = agg(hq, g1)
    hsum = jnp.sum(t1.reshape(R, K, D), axis=1)
    g2 = hop(hsum, t2, ts2, r1_ref[...], K * K)
    o2 = agg(hsum, g2)
    out_ref[:, 0:256] = o2
    out_ref[:, 256:512] = o1
    out_ref[:, 512:640] = hq


@functools.lru_cache(maxsize=None)
def _dense_call(R, interpret=False):
    grid = (S // R,)
    const = lambda i: (0, 0)
    row = lambda i: (i, 0)
    return pl.pallas_call(
        _dense_body,
        grid=grid,
        in_specs=[
            pl.BlockSpec((R, D), row),
            pl.BlockSpec((R * K, D), row),
            pl.BlockSpec((R * K * K, D), row),
            pl.BlockSpec((R * K, 1), row),
            pl.BlockSpec((R * K * K, 1), row),
            pl.BlockSpec((N_REL, D), const),
            pl.BlockSpec((2 * D, D), const),
            pl.BlockSpec((D, D), const),
            pl.BlockSpec((D, 1), const),
            pl.BlockSpec((D, D), const),
            pl.BlockSpec((1, D), const),
            pl.BlockSpec((2 * D, 2 * D), const),
            pl.BlockSpec((1, 2 * D), const),
            pl.BlockSpec((2 * D, 2 * D), const),
            pl.BlockSpec((1, 2 * D), const),
        ],
        out_specs=pl.BlockSpec((R, 640), row),
        out_shape=jax.ShapeDtypeStruct((S, 640), jnp.float32),
        interpret=interpret,
    )


def kernel(entity_embs, relation_embs, A1, A2, A3, Wx_w, Wx_b, W1_w, W1_b,
           W2_w, W2_b, u1, u2, c, adj_entity, adj_relation):
    i32 = jnp.int32
    adj_e = adj_entity.astype(i32).reshape(-1)
    adj_r = adj_relation.astype(i32).reshape(-1)
    seeds = jnp.concatenate([c, u1, u2]).astype(i32)

    hq_rows, t1_rows, r0, r1, e2 = _sc_stage_a()(adj_e, adj_r,
                                                 entity_embs, seeds)
    t2_rows = _sc_gather(S * K * K, D, "float32", 256)(entity_embs, e2)

    out640 = _dense_call(256)(
        hq_rows, t1_rows, t2_rows, r0.reshape(S * K, 1),
        r1.reshape(S * K * K, 1),
        relation_embs, A1, A2, A3, Wx_w, Wx_b.reshape(1, D),
        W1_w, W1_b.reshape(1, 2 * D), W2_w, W2_b.reshape(1, 2 * D))

    return jnp.concatenate([out640[0:B], out640[B:2 * B], out640[2 * B:]],
                           axis=-1)


# speedup vs baseline: 1.0503x; 1.0171x over previous
import functools

import jax
import jax.numpy as jnp
from jax import lax
from jax.experimental import pallas as pl
from jax.experimental.pallas import tpu as pltpu
from jax.experimental.pallas import tpu_sc as plsc

N_ENT = 100000
N_REL = 64
D = 128
K = 8
B = 1024
S = 3 * B

_NW = 32


@functools.lru_cache(maxsize=None)
def _sc_gather(M, Dw, dtype_name, chunk):
    dtype = jnp.dtype(dtype_name)
    bpw = M // _NW
    assert M % _NW == 0 and bpw % chunk == 0 and chunk % 8 == 0
    nch = bpw // chunk
    mesh = plsc.VectorSubcoreMesh(core_axis_name="c", subcore_axis_name="s")

    NB = 3

    def body(table_hbm, idx_hbm, out_hbm, idx_v, rows_v,
             g0, g1, g2, w0, w1, w2):
        wid = lax.axis_index("s") * 2 + lax.axis_index("c")
        base = wid * bpw
        pltpu.sync_copy(idx_hbm.at[pl.ds(base, bpw)], idx_v)
        gsem = (g0, g1, g2)
        wsem = (w0, w1, w2)

        def gstart(ci):
            return pltpu.async_copy(
                table_hbm.at[idx_v.at[pl.ds(ci * chunk, chunk)]],
                rows_v.at[ci % NB], gsem[ci % NB])

        gcop = [None] * nch
        wcop = [None] * nch
        for ci in range(min(NB, nch)):
            gcop[ci] = gstart(ci)
        for ci in range(nch):
            gcop[ci].wait()
            wcop[ci] = pltpu.async_copy(
                rows_v.at[ci % NB],
                out_hbm.at[pl.ds(base + ci * chunk, chunk)], wsem[ci % NB])
            nxt = ci + NB
            if nxt < nch:
                wcop[ci].wait()
                gcop[nxt] = gstart(nxt)
        for ci in range(nch):
            if ci + NB >= nch:
                wcop[ci].wait()

    return pl.kernel(
        body,
        out_type=jax.ShapeDtypeStruct((M, Dw), dtype),
        mesh=mesh,
        scratch_types=[
            pltpu.VMEM((bpw,), jnp.int32),
            pltpu.VMEM((NB, chunk, Dw), dtype),
            pltpu.SemaphoreType.DMA,
            pltpu.SemaphoreType.DMA,
            pltpu.SemaphoreType.DMA,
            pltpu.SemaphoreType.DMA,
            pltpu.SemaphoreType.DMA,
            pltpu.SemaphoreType.DMA,
        ],
    )


_SB = S // _NW


@functools.lru_cache(maxsize=None)
def _sc_stage_a():
    f32, i32 = jnp.float32, jnp.int32
    mesh = plsc.VectorSubcoreMesh(core_axis_name="c", subcore_axis_name="s")

    _dn = lax.GatherDimensionNumbers(offset_dims=(), collapsed_slice_dims=(0,),
                                     start_index_map=(0,))

    def vperm(vec, idx):
        return lax.gather(vec, idx[:, None], _dn, (1,),
                          mode=lax.GatherScatterMode.PROMISE_IN_BOUNDS)

    def word_idx(src_ref, n, dst_ref):
        iota = lax.broadcasted_iota(i32, (16,), 0)
        half = iota >> 3
        low = iota & 7
        for g in range(n // 16):
            s16 = src_ref[pl.ds(g * 16, 16)]
            for q in range(8):
                srep = vperm(s16, half + 2 * q)
                dst_ref[pl.ds(g * 128 + q * 16, 16)] = srep * K + low

    def body(adj_e, adj_r, embs, seeds, hq_o, t1_o, r0_o, r1_o, e2_o,
             sv, iw, e1v, r0v, iw2, e2v, r1v, fbuf,
             sema, semh, sem0, sem1):
        w = lax.axis_index("s") * 2 + lax.axis_index("c")
        s0 = w * _SB
        pltpu.sync_copy(seeds.at[pl.ds(s0, _SB)], sv)
        word_idx(sv, _SB, iw)
        cp_e = pltpu.async_copy(adj_e.at[iw], e1v, sema)
        cp_r = pltpu.async_copy(adj_r.at[iw], r0v, semh)
        cp_e.wait()
        cp_r.wait()
        pltpu.sync_copy(r0v, r0_o.at[pl.ds(s0 * K, _SB * K)])
        t1_cp0 = pltpu.async_copy(embs.at[e1v.at[pl.ds(0, 192)]],
                                  fbuf.at[0], sem0)
        word_idx(e1v, _SB * K, iw2)
        cp_e = pltpu.async_copy(adj_e.at[iw2], e2v, sema)
        cp_r = pltpu.async_copy(adj_r.at[iw2], r1v, sem1)
        cp_e.wait()
        cp_r.wait()
        pltpu.sync_copy(r1v, r1_o.at[pl.ds(s0 * K * K, _SB * K * K)])
        pltpu.sync_copy(e2v, e2_o.at[pl.ds(s0 * K * K, _SB * K * K)])
        sems = (sem0, sem1)
        cps = [t1_cp0, None, None, None]
        for ci in range(4):
            if ci + 1 < 4:
                cps[ci + 1] = pltpu.async_copy(
                    embs.at[e1v.at[pl.ds((ci + 1) * 192, 192)]],
                    fbuf.at[(ci + 1) % 2], sems[(ci + 1) % 2])
            cps[ci].wait()
            pltpu.sync_copy(fbuf.at[ci % 2],
                            t1_o.at[pl.ds(s0 * K + ci * 192, 192)])
        pltpu.async_copy(embs.at[sv], fbuf.at[0, pl.ds(0, _SB)], sem0).wait()
        pltpu.sync_copy(fbuf.at[0, pl.ds(0, _SB)], hq_o.at[pl.ds(s0, _SB)])

    return pl.kernel(
        body,
        out_type=(
            jax.ShapeDtypeStruct((S, D), f32),
            jax.ShapeDtypeStruct((S * K, D), f32),
            jax.ShapeDtypeStruct((S * K,), i32),
            jax.ShapeDtypeStruct((S * K * K,), i32),
            jax.ShapeDtypeStruct((S * K * K,), i32),
        ),
        mesh=mesh,
        scratch_types=[
            pltpu.VMEM((_SB,), i32),
            pltpu.VMEM((_SB * K,), i32),
            pltpu.VMEM((_SB * K,), i32),
            pltpu.VMEM((_SB * K,), i32),
            pltpu.VMEM((_SB * K * K,), i32),
            pltpu.VMEM((_SB * K * K,), i32),
            pltpu.VMEM((_SB * K * K,), i32),
            pltpu.VMEM((2, 192, D), f32),
            pltpu.SemaphoreType.DMA,
            pltpu.SemaphoreType.DMA,
            pltpu.SemaphoreType.DMA,
            pltpu.SemaphoreType.DMA,
        ],
    )


def _dense_body(hq_ref, t1_ref, t2_ref, r0_ref, r1_ref, rel_ref, A1_ref,
                A2_ref, A3_ref, Wxw_ref, Wxb_ref, W1w_ref, W1b_ref, W2w_ref,
                W2b_ref, out_ref):
    f32 = jnp.float32
    R = hq_ref.shape[0]
    ones_col = jnp.ones((D, 1), f32)

    def row_scale(v):
        n2 = jnp.dot(v * v, ones_col, preferred_element_type=f32)
        return jnp.minimum(lax.rsqrt(n2), 1.0)

    def norm_rows(v):
        return v * row_scale(v)

    def leaky(x):
        return jnp.where(x > 0, x, 0.2 * x)

    rel_n = norm_rows(rel_ref[...])
    A1h = A1_ref[0:D, :]
    A1r = A1_ref[D:2 * D, :]
    Brel = jnp.dot(rel_n, A1r, preferred_element_type=f32)
    A2 = A2_ref[...]
    A3c = A3_ref[...]
    Wxw = Wxw_ref[...]
    Wxb = Wxb_ref[...]
    W1f = W1w_ref[0:D, :] + W1w_ref[D:2 * D, :]
    W2f = W2w_ref[0:D, :] + W2w_ref[D:2 * D, :]
    W1b = W1b_ref[...]
    W2b = W2b_ref[...]

    hq = norm_rows(hq_ref[...])
    t1 = norm_rows(t1_ref[...])
    t2 = t2_ref[...]
    ts2 = row_scale(t2)

    def hop(h, tt, tscale, ridx, k):
        a = jnp.dot(h, A1h, preferred_element_type=f32)
        arep = jnp.broadcast_to(a.reshape(R, 1, D), (R, k, D)).reshape(R * k, D)
        oh = (ridx == lax.broadcasted_iota(jnp.int32, (R * k, N_REL), 1)
              ).astype(f32)
        b = jnp.dot(oh, Brel, preferred_element_type=f32)
        v = jnp.maximum(arep + b, 0.0)
        w = jnp.maximum(jnp.dot(v, A2, preferred_element_type=f32), 0.0)
        aw = jnp.dot(w, A3c, preferred_element_type=f32)
        ex = jnp.exp(1.0 / (1.0 + jnp.exp(-aw)))
        den = jnp.sum(ex.reshape(R, k, 1), axis=1)
        exw = ex * tscale if tscale is not None else ex
        num = jnp.sum((exw * tt).reshape(R, k, D), axis=1)
        e = num / den
        return leaky(jnp.dot(e, Wxw, preferred_element_type=f32) + Wxb)

    def agg(h, g):
        return (leaky(jnp.dot(h + g, W1f, preferred_element_type=f32) + W1b)
                + leaky(jnp.dot(h * g, W2f, preferred_element_type=f32) + W2b))

    g1 = hop(hq, t1, None, r0_ref[...], K)
    o1 = agg(hq, g1)
    hsum = jnp.sum(t1.reshape(R, K, D), axis=1)
    g2 = hop(hsum, t2, ts2, r1_ref[...], K * K)
    o2 = agg(hsum, g2)
    out_ref[:, 0:256] = o2
    out_ref[:, 256:512] = o1
    out_ref[:, 512:640] = hq


@functools.lru_cache(maxsize=None)
def _dense_call(R, interpret=False):
    grid = (S // R,)
    const = lambda i: (0, 0)
    row = lambda i: (i, 0)
    return pl.pallas_call(
        _dense_body,
        grid=grid,
        in_specs=[
            pl.BlockSpec((R, D), row),
            pl.BlockSpec((R * K, D), row),
            pl.BlockSpec((R * K * K, D), row),
            pl.BlockSpec((R * K, 1), row),
            pl.BlockSpec((R * K * K, 1), row),
            pl.BlockSpec((N_REL, D), const),
            pl.BlockSpec((2 * D, D), const),
            pl.BlockSpec((D, D), const),
            pl.BlockSpec((D, 1), const),
            pl.BlockSpec((D, D), const),
            pl.BlockSpec((1, D), const),
            pl.BlockSpec((2 * D, 2 * D), const),
            pl.BlockSpec((1, 2 * D), const),
            pl.BlockSpec((2 * D, 2 * D), const),
            pl.BlockSpec((1, 2 * D), const),
        ],
        out_specs=pl.BlockSpec((R, 640), lambda i: (i % (B // R), i // (B // R))),
        out_shape=jax.ShapeDtypeStruct((B, 1920), jnp.float32),
        interpret=interpret,
    )


def kernel(entity_embs, relation_embs, A1, A2, A3, Wx_w, Wx_b, W1_w, W1_b,
           W2_w, W2_b, u1, u2, c, adj_entity, adj_relation):
    i32 = jnp.int32
    adj_e = adj_entity.astype(i32).reshape(-1)
    adj_r = adj_relation.astype(i32).reshape(-1)
    seeds = jnp.concatenate([c, u1, u2]).astype(i32)

    hq_rows, t1_rows, r0, r1, e2 = _sc_stage_a()(adj_e, adj_r,
                                                 entity_embs, seeds)
    t2_rows = _sc_gather(S * K * K, D, "float32", 256)(entity_embs, e2)

    return _dense_call(256)(
        hq_rows, t1_rows, t2_rows, r0.reshape(S * K, 1),
        r1.reshape(S * K * K, 1),
        relation_embs, A1, A2, A3, Wx_w, Wx_b.reshape(1, D),
        W1_w, W1_b.reshape(1, 2 * D), W2_w, W2_b.reshape(1, 2 * D))
